# baseline (speedup 1.0000x reference)
import functools

import jax
import jax.numpy as jnp
from jax import lax
from jax.experimental import pallas as pl
from jax.experimental.pallas import tpu as pltpu
from jax.experimental.pallas import tpu_sc as plsc

NC = 2
NS = 16
NW = NC * NS

G_STATIC = 128


def _fill_vec(buf, vec, start, nv):
  def body(i, carry):
    buf[pl.ds(i * 16, 16)] = vec
    return carry
  lax.fori_loop(start, nv, body, 0)


def _sc_impl(E_BASE, E_EXP, N, ei, ea, bv, ee, wexp, wvn, wseg, aux16):
  NV = 4
  ATTR_BASE_W = E_BASE * 16
  ATTR_W_PER = ATTR_BASE_W // NW
  EXP_W_PER = E_EXP * 16 // NW
  IDX_PER = E_BASE // NW
  PAT = 20000
  RING = 20000
  T0_PER = E_BASE // NW
  T1_PER = E_EXP // NW
  NT2 = 2 * NV * N
  T2_CHUNK = ((NT2 + NW - 1) // NW + 7) // 8 * 8
  T2_NFULL = NT2 // T2_CHUNK
  T2_REM = NT2 - T2_NFULL * T2_CHUNK
  PAIRS_PER = E_EXP // NW
  DROUNDS = [(k * 3200, 3200, 200) for k in range(7)] + [(22400, 2600, 163)]
  HALF0 = 24992
  HALF1 = N - HALF0
  VH_W = G_STATIC * 128

  O_IDX = 2 * (E_BASE + E_EXP)
  O_ATTR = (E_BASE + E_EXP) * 16
  O_TYPES = E_BASE + E_EXP + 2 * NV * N
  O_VEI = 2 * (2 * NV * N)
  O_VATTR = 2 * NV * N * 16

  mesh = plsc.VectorSubcoreMesh(
      core_axis_name="c", subcore_axis_name="s", num_cores=NC, num_subcores=NS)

  @functools.partial(
      pl.kernel,
      out_type=(
          jax.ShapeDtypeStruct((O_IDX,), jnp.int32),
          jax.ShapeDtypeStruct((O_ATTR,), jnp.int32),
          jax.ShapeDtypeStruct((O_TYPES,), jnp.int32),
          jax.ShapeDtypeStruct((NV * VH_W,), jnp.int32),
          jax.ShapeDtypeStruct((O_VEI,), jnp.int32),
          jax.ShapeDtypeStruct((O_VATTR,), jnp.int32),
      ),
      mesh=mesh,
      compiler_params=pltpu.CompilerParams(needs_layout_passes=False),
      scratch_types=[
          pltpu.VMEM((PAT,), jnp.int32),
          pltpu.VMEM((PAT,), jnp.int32),
          pltpu.VMEM((6400,), jnp.int32),
          pltpu.VMEM((3200,), jnp.int32),
          pltpu.VMEM((3200,), jnp.int32),
          pltpu.VMEM((HALF1,), jnp.int32),
          pltpu.VMEM((RING,), jnp.int32),
          pltpu.VMEM((RING,), jnp.int32),
          pltpu.VMEM((16,), jnp.int32),
          pltpu.SemaphoreType.DMA,
          pltpu.SemaphoreType.DMA,
          pltpu.SemaphoreType.DMA,
          pltpu.SemaphoreType.DMA,
          pltpu.SemaphoreType.DMA,
      ],
  )
  def body(ei, ea, bv, ee, wexp, wvn, wseg, aux16,
           o_idx, o_attr, o_types, o_vh, o_vei, o_vattr,
           pat_f, cbuf, dbin, dbs, dbd, vbuf, bb0, bb1, auxv,
           sem_pat, sem_bg, sem_d, sem_in, sem_out):
    wid = lax.axis_index("s") * NC + lax.axis_index("c")
    drain = []

    pltpu.sync_copy(wexp, pat_f.at[pl.ds(0, 16)])
    _fill_vec(pat_f, pat_f[pl.ds(0, 16)], 1, PAT // 16)
    p2_outs = []
    for j in range(EXP_W_PER // PAT):
      off = ATTR_BASE_W + wid * EXP_W_PER + j * PAT
      p2_outs.append(
          pltpu.async_copy(pat_f, o_attr.at[pl.ds(off, PAT)], sem_pat))

    _fill_vec(cbuf, jnp.zeros((16,), jnp.int32), 0, 500)
    _fill_vec(cbuf, jnp.full((16,), 1, jnp.int32), 500, 1000)
    _fill_vec(cbuf, jnp.full((16,), 2, jnp.int32), 1000, 1250)
    base = wid * T0_PER
    for j in range(6):
      drain.append(pltpu.async_copy(
          cbuf.at[pl.ds(0, 8000)],
          o_types.at[pl.ds(base + j * 8000, 8000)], sem_bg))
    drain.append(pltpu.async_copy(
        cbuf.at[pl.ds(0, 2000)], o_types.at[pl.ds(base + 48000, 2000)], sem_bg))
    base = E_BASE + wid * T1_PER
    for j in range(3):
      drain.append(pltpu.async_copy(
          cbuf.at[pl.ds(8000, 8000)],
          o_types.at[pl.ds(base + j * 8000, 8000)], sem_bg))
    drain.append(pltpu.async_copy(
        cbuf.at[pl.ds(8000, 1000)],
        o_types.at[pl.ds(base + 24000, 1000)], sem_bg))
    base = E_BASE + E_EXP

    @pl.when(wid < T2_NFULL)
    def _():
      b2 = base + wid * T2_CHUNK
      for j in range(3):
        pltpu.async_copy(cbuf.at[pl.ds(16000, 4000)],
                         o_types.at[pl.ds(b2 + j * 4000, 4000)], sem_bg).wait()
      pltpu.async_copy(cbuf.at[pl.ds(16000, T2_CHUNK - 12000)],
                       o_types.at[pl.ds(b2 + 12000, T2_CHUNK - 12000)],
                       sem_bg).wait()

    @pl.when(wid == T2_NFULL)
    def _():
      b2 = base + T2_NFULL * T2_CHUNK
      for j in range(3):
        pltpu.async_copy(cbuf.at[pl.ds(16000, 4000)],
                         o_types.at[pl.ds(b2 + j * 4000, 4000)], sem_bg).wait()
      pltpu.async_copy(cbuf.at[pl.ds(16000, T2_REM - 12000)],
                       o_types.at[pl.ds(b2 + 12000, T2_REM - 12000)],
                       sem_bg).wait()

    iota2 = lax.iota(jnp.int32, 16) * 2
    pbase = wid * PAIRS_PER
    prev = []
    for (poff, npairs, nvregs) in DROUNDS:
      pltpu.sync_copy(ee.at[pl.ds((pbase + poff) * 2, npairs * 2)],
                      dbin.at[pl.ds(0, npairs * 2)])
      for d in prev:
        d.wait()
      prev = []

      def deint(j, carry):
        idx = iota2 + j * 32
        dbs[pl.ds(j * 16, 16)] = plsc.load_gather(dbin, [idx])
        dbd[pl.ds(j * 16, 16)] = plsc.load_gather(dbin, [idx + 1])
        return carry

      lax.fori_loop(0, nvregs, deint, 0)
      prev.append(pltpu.async_copy(
          dbs.at[pl.ds(0, npairs)],
          o_idx.at[pl.ds(E_BASE + pbase + poff, npairs)], sem_d))
      prev.append(pltpu.async_copy(
          dbd.at[pl.ds(0, npairs)],
          o_idx.at[pl.ds(2 * E_BASE + E_EXP + pbase + poff, npairs)], sem_d))
    drain.extend(prev)

    pltpu.sync_copy(aux16, auxv)
    p = wid // 2
    h = wid % 2
    off_iota = jnp.where(p < 4, p * 2 * N, 2 * NV * N + (2 * p - 7) * N)
    j = p - 8
    off_bv = jnp.where(j < 4, (2 * j + 1) * N, 2 * NV * N + (2 * j - 8) * N)
    k = jnp.maximum(j, 0) % 4

    def gen_iota(hoff, sz, nv, out_off):
      def fill(i, carry):
        vbuf[pl.ds(i * 16, 16)] = lax.iota(jnp.int32, 16) + (hoff + i * 16)
        return carry
      lax.fori_loop(0, nv, fill, 0)
      pltpu.async_copy(vbuf.at[pl.ds(0, sz)],
                       o_vei.at[pl.ds(out_off + hoff, sz)], sem_bg).wait()

    def gen_bv(hoff, sz, nv, out_off):
      cvec = plsc.load_gather(auxv, [jnp.zeros((16,), jnp.int32) + k])
      pltpu.sync_copy(bv.at[pl.ds(hoff, sz)], vbuf.at[pl.ds(0, sz)])

      def addc(i, carry):
        vbuf[pl.ds(i * 16, 16)] = vbuf[pl.ds(i * 16, 16)] + cvec
        return carry
      lax.fori_loop(0, nv, addc, 0)
      pltpu.async_copy(vbuf.at[pl.ds(0, sz)],
                       o_vei.at[pl.ds(out_off + hoff, sz)], sem_bg).wait()

    @pl.when((p < 8) & (h == 0))
    def _():
      gen_iota(0, HALF0, HALF0 // 16, off_iota)

    @pl.when((p < 8) & (h == 1))
    def _():
      gen_iota(HALF0, HALF1, HALF1 // 16, off_iota)

    @pl.when((p >= 8) & (h == 0))
    def _():
      gen_bv(0, HALF0, HALF0 // 16, off_bv)

    @pl.when((p >= 8) & (h == 1))
    def _():
      gen_bv(HALF0, HALF1, HALF1 // 16, off_bv)

    for d in p2_outs:
      d.wait()
    seg = wid // 4
    q = wid % 4
    VA_SEG_W = N * 16
    VA_Q_W = VA_SEG_W // 4
    pltpu.sync_copy(wseg.at[pl.ds(seg * 16, 16)], pat_f.at[pl.ds(0, 16)])
    _fill_vec(pat_f, pat_f[pl.ds(0, 16)], 1, PAT // 16)
    p7_outs = []
    for j in range(VA_Q_W // PAT):
      off = seg * VA_SEG_W + q * VA_Q_W + j * PAT
      p7_outs.append(
          pltpu.async_copy(pat_f, o_vattr.at[pl.ds(off, PAT)], sem_pat))

    for d in p7_outs:
      d.wait()

    @pl.when(wid < NV)
    def _():
      pltpu.sync_copy(wvn.at[pl.ds(wid * 128, 128)], pat_f.at[pl.ds(0, 128)])
      vs = [pat_f[pl.ds(r * 16, 16)] for r in range(8)]

      def repl(i, carry):
        for r in range(8):
          pat_f[pl.ds(i * 128 + r * 16, 16)] = vs[r]
        return carry

      lax.fori_loop(1, VH_W // 128, repl, 0)
      pltpu.async_copy(pat_f.at[pl.ds(0, VH_W)],
                       o_vh.at[pl.ds(wid * VH_W, VH_W)], sem_pat).wait()

    jobs = []
    for jj in range(ATTR_W_PER // RING):
      jobs.append((ea, jj * RING + wid * ATTR_W_PER,
                   o_attr, jj * RING + wid * ATTR_W_PER, RING))
    for (soff, sz) in ((0, RING), (RING, RING), (2 * RING, IDX_PER - 2 * RING)):
      jobs.append((ei, wid * IDX_PER + soff,
                   o_idx, wid * IDX_PER + soff, sz))
      jobs.append((ei, E_BASE + wid * IDX_PER + soff,
                   o_idx, (E_BASE + E_EXP) + wid * IDX_PER + soff, sz))
    bbs = [bb0, bb1]
    n = len(jobs)
    d_in = [None] * n
    d_out = [None] * n

    def start_in(i):
      src, soff, _, _, sz = jobs[i]
      return pltpu.async_copy(src.at[pl.ds(soff, sz)],
                              bbs[i % 2].at[pl.ds(0, sz)], sem_in)

    d_in[0] = start_in(0)
    for i in range(n):
      if i + 1 < n:
        if i >= 1:
          d_out[i - 1].wait()
        d_in[i + 1] = start_in(i + 1)
      d_in[i].wait()
      _, _, dst, doff, sz = jobs[i]
      d_out[i] = pltpu.async_copy(bbs[i % 2].at[pl.ds(0, sz)],
                                  dst.at[pl.ds(doff, sz)], sem_out)
    d_out[n - 2].wait()
    d_out[n - 1].wait()

    for d in drain:
      d.wait()

  return body(ei, ea, bv, ee, wexp, wvn, wseg, aux16)


def kernel(edge_index, edge_attr, batch_vec, expander_edges, num_graphs,
           exp_edge_attr_weight, virt_node_emb_weight,
           virt_edge_in_emb_weight, virt_edge_out_emb_weight):
  E_BASE = edge_index.shape[1]
  E_EXP = expander_edges.shape[0]
  N = batch_vec.shape[0]
  NV = virt_node_emb_weight.shape[0]

  def as_i32(x):
    return lax.bitcast_convert_type(x, jnp.int32).reshape(-1)

  ei = edge_index.reshape(-1)
  ea = as_i32(edge_attr)
  ee = expander_edges.reshape(-1)
  wexp = as_i32(exp_edge_attr_weight)
  wvn = as_i32(virt_node_emb_weight)
  wseg = as_i32(jnp.stack(
      [virt_edge_in_emb_weight, virt_edge_out_emb_weight], axis=1))
  c4 = N + jnp.arange(NV, dtype=jnp.int32) * num_graphs
  aux16 = jnp.concatenate([c4, jnp.zeros((16 - NV,), jnp.int32)])

  o_idx, o_attr, o_types, o_vh, o_vei, o_vattr = _sc_impl(
      E_BASE, E_EXP, N, ei, ea, batch_vec, ee, wexp, wvn, wseg, aux16)

  def as_f32(x):
    return lax.bitcast_convert_type(x, jnp.float32)

  return (
      o_idx.reshape(2, E_BASE + E_EXP),
      as_f32(o_attr.reshape(E_BASE + E_EXP, 16)),
      o_types,
      as_f32(o_vh.reshape(NV * G_STATIC, 128)),
      o_vei.reshape(2, 2 * NV * N),
      as_f32(o_vattr.reshape(2 * NV * N, 16)),
  )


# speedup vs baseline: 1.0581x; 1.0581x over previous
import functools

import jax
import jax.numpy as jnp
from jax import lax
from jax.experimental import pallas as pl
from jax.experimental.pallas import tpu as pltpu
from jax.experimental.pallas import tpu_sc as plsc

NC = 2
NS = 16
NW = NC * NS

G_STATIC = 128


def _fill_vec(buf, vec, start, nv):
  def body(i, carry):
    buf[pl.ds(i * 16, 16)] = vec
    return carry
  lax.fori_loop(start, nv, body, 0)


def _sc_impl(E_BASE, E_EXP, N, ei, ea, bv, ee, wexp, wvn, wseg, aux16):
  NV = 4
  ATTR_BASE_W = E_BASE * 16
  ATTR_W_PER = ATTR_BASE_W // NW
  EXP_W_PER = E_EXP * 16 // NW
  IDX_PER = E_BASE // NW
  PAT = 20000
  RING = 20000
  IRING = 12504
  ICHUNKS = ((0, 12504), (12504, 12504), (25008, 12504), (37512, 12488))
  T0_PER = E_BASE // NW
  T1_PER = E_EXP // NW
  NT2 = 2 * NV * N
  T2_CHUNK = ((NT2 + NW - 1) // NW + 7) // 8 * 8
  T2_NFULL = NT2 // T2_CHUNK
  T2_REM = NT2 - T2_NFULL * T2_CHUNK
  PAIRS_PER = E_EXP // NW
  DROUNDS = [(k * 3200, 3200, 200) for k in range(7)] + [(22400, 2600, 163)]
  VEI_SUB0 = ((0, 12496, 781), (12496, 12496, 781))
  VEI_SUB1 = ((24992, 12496, 781), (37488, 12512, 782))
  VH_W = G_STATIC * 128

  O_IDX = 2 * (E_BASE + E_EXP)
  O_ATTR = (E_BASE + E_EXP) * 16
  O_TYPES = E_BASE + E_EXP + 2 * NV * N
  O_VEI = 2 * (2 * NV * N)
  O_VATTR = 2 * NV * N * 16

  mesh = plsc.VectorSubcoreMesh(
      core_axis_name="c", subcore_axis_name="s", num_cores=NC, num_subcores=NS)

  @functools.partial(
      pl.kernel,
      out_type=(
          jax.ShapeDtypeStruct((O_IDX,), jnp.int32),
          jax.ShapeDtypeStruct((O_ATTR,), jnp.float32),
          jax.ShapeDtypeStruct((O_TYPES,), jnp.int32),
          jax.ShapeDtypeStruct((NV * VH_W,), jnp.float32),
          jax.ShapeDtypeStruct((O_VEI,), jnp.int32),
          jax.ShapeDtypeStruct((O_VATTR,), jnp.float32),
      ),
      mesh=mesh,
      compiler_params=pltpu.CompilerParams(needs_layout_passes=False),
      scratch_types=[
          pltpu.VMEM((PAT,), jnp.float32),
          pltpu.VMEM((12000,), jnp.int32),
          pltpu.VMEM((IRING + 12,), jnp.int32),
          pltpu.VMEM((3200,), jnp.int32),
          pltpu.VMEM((3200,), jnp.int32),
          pltpu.VMEM((IRING + 12,), jnp.int32),
          pltpu.VMEM((RING,), jnp.float32),
          pltpu.VMEM((RING,), jnp.float32),
          pltpu.VMEM((16,), jnp.int32),
          pltpu.SemaphoreType.DMA,
          pltpu.SemaphoreType.DMA,
          pltpu.SemaphoreType.DMA,
          pltpu.SemaphoreType.DMA,
          pltpu.SemaphoreType.DMA,
          pltpu.SemaphoreType.DMA,
      ],
  )
  def body(ei, ea, bv, ee, wexp, wvn, wseg, aux16,
           o_idx, o_attr, o_types, o_vh, o_vei, o_vattr,
           pat_f, cbuf, dbin, dbs, dbd, vbuf, bb0, bb1, auxv,
           sem_pat, sem_bg, sem_d, sem_v, sem_in, sem_out):
    wid = lax.axis_index("s") * NC + lax.axis_index("c")
    drain = []

    pltpu.sync_copy(wexp, pat_f.at[pl.ds(0, 16)])
    _fill_vec(pat_f, pat_f[pl.ds(0, 16)], 1, PAT // 16)
    p2_outs = []
    for j in range(EXP_W_PER // PAT):
      off = ATTR_BASE_W + wid * EXP_W_PER + j * PAT
      p2_outs.append(
          pltpu.async_copy(pat_f, o_attr.at[pl.ds(off, PAT)], sem_pat))

    _fill_vec(cbuf, jnp.zeros((16,), jnp.int32), 0, 375)
    _fill_vec(cbuf, jnp.full((16,), 1, jnp.int32), 375, 625)
    _fill_vec(cbuf, jnp.full((16,), 2, jnp.int32), 625, 750)
    base = wid * T0_PER
    for j in range(8):
      drain.append(pltpu.async_copy(
          cbuf.at[pl.ds(0, 6000)],
          o_types.at[pl.ds(base + j * 6000, 6000)], sem_bg))
    drain.append(pltpu.async_copy(
        cbuf.at[pl.ds(0, 2000)], o_types.at[pl.ds(base + 48000, 2000)], sem_bg))
    base = E_BASE + wid * T1_PER
    for j in range(6):
      drain.append(pltpu.async_copy(
          cbuf.at[pl.ds(6000, 4000)],
          o_types.at[pl.ds(base + j * 4000, 4000)], sem_bg))
    drain.append(pltpu.async_copy(
        cbuf.at[pl.ds(6000, 1000)],
        o_types.at[pl.ds(base + 24000, 1000)], sem_bg))
    base = E_BASE + E_EXP

    @pl.when(wid < T2_NFULL)
    def _():
      b2 = base + wid * T2_CHUNK
      for j in range(6):
        pltpu.async_copy(cbuf.at[pl.ds(10000, 2000)],
                         o_types.at[pl.ds(b2 + j * 2000, 2000)], sem_bg).wait()
      pltpu.async_copy(cbuf.at[pl.ds(10000, T2_CHUNK - 12000)],
                       o_types.at[pl.ds(b2 + 12000, T2_CHUNK - 12000)],
                       sem_bg).wait()

    @pl.when(wid == T2_NFULL)
    def _():
      b2 = base + T2_NFULL * T2_CHUNK
      for j in range(6):
        pltpu.async_copy(cbuf.at[pl.ds(10000, 2000)],
                         o_types.at[pl.ds(b2 + j * 2000, 2000)], sem_bg).wait()
      pltpu.async_copy(cbuf.at[pl.ds(10000, T2_REM - 12000)],
                       o_types.at[pl.ds(b2 + 12000, T2_REM - 12000)],
                       sem_bg).wait()

    iota2 = lax.iota(jnp.int32, 16) * 2
    pbase = wid * PAIRS_PER
    prev = []
    for (poff, npairs, nvregs) in DROUNDS:
      pltpu.sync_copy(ee.at[pl.ds((pbase + poff) * 2, npairs * 2)],
                      dbin.at[pl.ds(0, npairs * 2)])
      for d in prev:
        d.wait()
      prev = []

      def deint(j, carry):
        idx = iota2 + j * 32
        dbs[pl.ds(j * 16, 16)] = plsc.load_gather(dbin, [idx])
        dbd[pl.ds(j * 16, 16)] = plsc.load_gather(dbin, [idx + 1])
        return carry

      lax.fori_loop(0, nvregs, deint, 0)
      prev.append(pltpu.async_copy(
          dbs.at[pl.ds(0, npairs)],
          o_idx.at[pl.ds(E_BASE + pbase + poff, npairs)], sem_d))
      prev.append(pltpu.async_copy(
          dbd.at[pl.ds(0, npairs)],
          o_idx.at[pl.ds(2 * E_BASE + E_EXP + pbase + poff, npairs)], sem_d))
    for d in prev:
      d.wait()

    pltpu.sync_copy(aux16, auxv)
    p = wid // 2
    h = wid % 2
    off_iota = jnp.where(p < 4, p * 2 * N, 2 * NV * N + (2 * p - 7) * N)
    j = p - 8
    off_bv = jnp.where(j < 4, (2 * j + 1) * N, 2 * NV * N + (2 * j - 8) * N)
    k = jnp.maximum(j, 0) % 4

    def gen_iota(sub, out_off):
      for (hoff, sz, nv) in sub:
        def fill(i, carry):
          vbuf[pl.ds(i * 16, 16)] = lax.iota(jnp.int32, 16) + (hoff + i * 16)
          return carry
        lax.fori_loop(0, nv, fill, 0)
        pltpu.async_copy(vbuf.at[pl.ds(0, sz)],
                         o_vei.at[pl.ds(out_off + hoff, sz)], sem_v).wait()

    def gen_bv(sub, out_off):
      cvec = plsc.load_gather(auxv, [jnp.zeros((16,), jnp.int32) + k])
      for (hoff, sz, nv) in sub:
        pltpu.sync_copy(bv.at[pl.ds(hoff, sz)], vbuf.at[pl.ds(0, sz)])

        def addc(i, carry):
          vbuf[pl.ds(i * 16, 16)] = vbuf[pl.ds(i * 16, 16)] + cvec
          return carry
        lax.fori_loop(0, nv, addc, 0)
        pltpu.async_copy(vbuf.at[pl.ds(0, sz)],
                         o_vei.at[pl.ds(out_off + hoff, sz)], sem_v).wait()

    @pl.when((p < 8) & (h == 0))
    def _():
      gen_iota(VEI_SUB0, off_iota)

    @pl.when((p < 8) & (h == 1))
    def _():
      gen_iota(VEI_SUB1, off_iota)

    @pl.when((p >= 8) & (h == 0))
    def _():
      gen_bv(VEI_SUB0, off_bv)

    @pl.when((p >= 8) & (h == 1))
    def _():
      gen_bv(VEI_SUB1, off_bv)

    for d in p2_outs:
      d.wait()
    seg = wid // 4
    q = wid % 4
    VA_SEG_W = N * 16
    VA_Q_W = VA_SEG_W // 4
    pltpu.sync_copy(wseg.at[pl.ds(seg * 16, 16)], pat_f.at[pl.ds(0, 16)])
    _fill_vec(pat_f, pat_f[pl.ds(0, 16)], 1, PAT // 16)
    p7_outs = []
    for j in range(VA_Q_W // PAT):
      off = seg * VA_SEG_W + q * VA_Q_W + j * PAT
      p7_outs.append(
          pltpu.async_copy(pat_f, o_vattr.at[pl.ds(off, PAT)], sem_pat))

    for d in p7_outs:
      d.wait()

    @pl.when(wid < NV)
    def _():
      pltpu.sync_copy(wvn.at[pl.ds(wid * 128, 128)], pat_f.at[pl.ds(0, 128)])
      vs = [pat_f[pl.ds(r * 16, 16)] for r in range(8)]

      def repl(i, carry):
        for r in range(8):
          pat_f[pl.ds(i * 128 + r * 16, 16)] = vs[r]
        return carry

      lax.fori_loop(1, VH_W // 128, repl, 0)
      pltpu.async_copy(pat_f.at[pl.ds(0, VH_W)],
                       o_vh.at[pl.ds(wid * VH_W, VH_W)], sem_pat).wait()

    ijobs = []
    for r in range(2):
      for (coff, csz) in ICHUNKS:
        ijobs.append((E_BASE * r + wid * IDX_PER + coff,
                      (E_BASE + E_EXP) * r + wid * IDX_PER + coff, csz))
    ibbs = [dbin, vbuf]
    ni = len(ijobs)
    i_in = [None] * ni
    i_out = [None] * ni

    def istart(i):
      soff, _, csz = ijobs[i]
      return pltpu.async_copy(ei.at[pl.ds(soff, csz)],
                              ibbs[i % 2].at[pl.ds(0, csz)], sem_in)

    i_in[0] = istart(0)
    for i in range(ni):
      if i + 1 < ni:
        if i >= 1:
          i_out[i - 1].wait()
        i_in[i + 1] = istart(i + 1)
      i_in[i].wait()
      _, doff, csz = ijobs[i]
      i_out[i] = pltpu.async_copy(ibbs[i % 2].at[pl.ds(0, csz)],
                                  o_idx.at[pl.ds(doff, csz)], sem_out)
    i_out[ni - 2].wait()
    i_out[ni - 1].wait()

    bbs = [bb0, bb1]
    n = ATTR_W_PER // RING

    def astart(i):
      off = wid * ATTR_W_PER + i * RING
      return pltpu.async_copy(ea.at[pl.ds(off, RING)],
                              bbs[i % 2], sem_in)

    a_out = [None] * n
    a_in = [None] * n
    a_in[0] = astart(0)
    for i in range(n):
      if i + 1 < n:
        if i >= 1:
          a_out[i - 1].wait()
        a_in[i + 1] = astart(i + 1)
      a_in[i].wait()
      off = wid * ATTR_W_PER + i * RING
      a_out[i] = pltpu.async_copy(bbs[i % 2],
                                  o_attr.at[pl.ds(off, RING)], sem_out)
    a_out[n - 2].wait()
    a_out[n - 1].wait()

    for d in drain:
      d.wait()

  return body(ei, ea, bv, ee, wexp, wvn, wseg, aux16)


def kernel(edge_index, edge_attr, batch_vec, expander_edges, num_graphs,
           exp_edge_attr_weight, virt_node_emb_weight,
           virt_edge_in_emb_weight, virt_edge_out_emb_weight):
  E_BASE = edge_index.shape[1]
  E_EXP = expander_edges.shape[0]
  N = batch_vec.shape[0]
  NV = virt_node_emb_weight.shape[0]

  ei = edge_index.reshape(-1)
  ea = edge_attr.reshape(-1)
  ee = expander_edges.reshape(-1)
  wexp = exp_edge_attr_weight.reshape(-1)
  wvn = virt_node_emb_weight.reshape(-1)
  wseg = jnp.stack(
      [virt_edge_in_emb_weight, virt_edge_out_emb_weight], axis=1).reshape(-1)
  c4 = N + jnp.arange(NV, dtype=jnp.int32) * num_graphs
  aux16 = jnp.concatenate([c4, jnp.zeros((16 - NV,), jnp.int32)])

  o_idx, o_attr, o_types, o_vh, o_vei, o_vattr = _sc_impl(
      E_BASE, E_EXP, N, ei, ea, batch_vec, ee, wexp, wvn, wseg, aux16)

  return (
      o_idx.reshape(2, E_BASE + E_EXP),
      o_attr.reshape(E_BASE + E_EXP, 16),
      o_types,
      o_vh.reshape(NV * G_STATIC, 128),
      o_vei.reshape(2, 2 * NV * N),
      o_vattr.reshape(2 * NV * N, 16),
  )
